# baseline (speedup 1.0000x reference)
import functools

import numpy as np
import jax
import jax.numpy as jnp
from jax import lax
from jax.experimental import pallas as pl
from jax.experimental.pallas import tpu as pltpu
from jax.experimental.pallas import tpu_sc as plsc


def _sinusoidal_pe(max_len, d):
    pos = np.arange(max_len, dtype=np.float32)[:, None]
    div = np.exp(np.arange(0, d, 2, dtype=np.float32) * (-np.log(10000.0) / d))
    pe = np.zeros((max_len, d), dtype=np.float32)
    pe[:, 0::2] = np.sin(pos * div)
    pe[:, 1::2] = np.cos(pos * div)
    return pe


def kernel(input, token_table):
    B, S = input.shape
    V, E = token_table.shape
    NC, NS = 2, 16
    NW = NC * NS
    L = 16
    W = B // NW
    TE = E // 8
    NBUF = 4
    LOOK = 2

    ids_t = input.astype(jnp.int32).T
    pe = jnp.asarray(_sinusoidal_pe(S, E))

    mesh = plsc.VectorSubcoreMesh(core_axis_name="c", subcore_axis_name="s")

    @functools.partial(
        pl.kernel,
        out_type=jax.ShapeDtypeStruct((S, TE, NW, 8, W), jnp.float32),
        mesh=mesh,
        compiler_params=pltpu.CompilerParams(
            use_tc_tiling_on_sc=False, needs_layout_passes=False),
        scratch_types=[
            pltpu.VMEM((S, W), jnp.int32),
            pltpu.VMEM((S, E), jnp.float32),
            pltpu.VMEM((NBUF * W, E), jnp.float32),
            pltpu.VMEM((NBUF * TE, 8, W), jnp.float32),
            pltpu.VMEM((W // L, L), jnp.int32),
        ] + [pltpu.SemaphoreType.DMA] * (2 * NBUF),
    )
    def run(table_hbm, ids_hbm, pe_hbm, out_hbm, idx_v, pe_v, buf_v, tbuf_v,
            rows_v, *sems):
        gsems = sems[:NBUF]
        ssems = sems[NBUF:]
        wid = lax.axis_index("s") * NC + lax.axis_index("c")
        def load_ids(s2, c2):
            pltpu.sync_copy(ids_hbm.at[s2, pl.ds(wid * W, W)], idx_v.at[s2])
            return c2

        lax.fori_loop(0, S, load_ids, 0)
        pltpu.sync_copy(pe_hbm, pe_v)
        for bc in range(W // L):
            rows_v[bc, :] = lax.iota(jnp.int32, L) + bc * L

        def gather(g, b):
            return pltpu.make_async_copy(
                table_hbm.at[idx_v.at[g]], buf_v.at[pl.ds(b * W, W)], gsems[b])

        def scatter(g, b):
            return pltpu.make_async_copy(
                tbuf_v.at[pl.ds(b * TE, TE)], out_hbm.at[g, :, wid], ssems[b])

        for g0 in range(LOOK):
            gather(g0, g0).start()

        def outer(i, carry):
            for b in range(NBUF):
                g = i * NBUF + b
                gather(g, b).wait()

                pec = [pe_v[g, pl.ds(c * L, L)] for c in range(E // L)]

                def add_row(r, c2):
                    for c in range(E // L):
                        sl = pl.ds(c * L, L)
                        buf_v[b * W + r, sl] = buf_v[b * W + r, sl] + pec[c]
                    return c2

                lax.fori_loop(0, W, add_row, 0)

                for bc in range(W // L):
                    rvec = rows_v[bc, :] + b * W

                    def xpose(te, c2, _bc=bc, _rvec=rvec):
                        for ep in range(8):
                            col = (_rvec * 0) + (te * 8 + ep)
                            v = plsc.load_gather(buf_v, [_rvec, col])
                            tbuf_v[b * TE + te, ep, pl.ds(_bc * L, L)] = v
                        return c2

                    lax.fori_loop(0, TE, xpose, 0)

                scatter(g, b).start()

                gn = g + LOOK
                nb = (b + LOOK) % NBUF

                @pl.when(gn < S)
                def _():
                    @pl.when(gn >= NBUF)
                    def _():
                        scatter(gn - NBUF, nb).wait()
                    gather(gn, nb).start()
            return carry

        lax.fori_loop(0, S // NBUF, outer, 0)

        for g0 in range(S - NBUF, S):
            scatter(g0, g0 % NBUF).wait()

    out5d = run(token_table, ids_t, pe)
    return out5d.transpose(2, 4, 0, 1, 3).reshape(B, S, E)


# speedup vs baseline: 1.2495x; 1.2495x over previous
import functools

import numpy as np
import jax
import jax.numpy as jnp
from jax import lax
from jax.experimental import pallas as pl
from jax.experimental.pallas import tpu as pltpu
from jax.experimental.pallas import tpu_sc as plsc


def _sinusoidal_pe(max_len, d):
    pos = np.arange(max_len, dtype=np.float32)[:, None]
    div = np.exp(np.arange(0, d, 2, dtype=np.float32) * (-np.log(10000.0) / d))
    pe = np.zeros((max_len, d), dtype=np.float32)
    pe[:, 0::2] = np.sin(pos * div)
    pe[:, 1::2] = np.cos(pos * div)
    return pe


def kernel(input, token_table):
    B, S = input.shape
    V, E = token_table.shape
    NC, NS = 2, 16
    NW = NC * NS
    L = 16
    W = B // NW
    TE = E // 8
    P = E + 1
    NBUF = 4
    LOOK = 2

    ids_t = input.astype(jnp.int32).T
    pe = jnp.asarray(_sinusoidal_pe(S, E))

    mesh = plsc.VectorSubcoreMesh(core_axis_name="c", subcore_axis_name="s")

    @functools.partial(
        pl.kernel,
        out_type=jax.ShapeDtypeStruct((S, TE, NW, 8, W), jnp.float32),
        mesh=mesh,
        compiler_params=pltpu.CompilerParams(
            use_tc_tiling_on_sc=False, needs_layout_passes=False),
        scratch_types=[
            pltpu.VMEM((S, W), jnp.int32),
            pltpu.VMEM((S, E), jnp.float32),
            pltpu.VMEM((NBUF * W, E), jnp.float32),
            pltpu.VMEM((W, P), jnp.float32),
            pltpu.VMEM((NBUF * TE, 8, W), jnp.float32),
            pltpu.VMEM((W // L, L), jnp.int32),
        ] + [pltpu.SemaphoreType.DMA] * (2 * NBUF + 1),
    )
    def run(table_hbm, ids_hbm, pe_hbm, out_hbm, idx_v, pe_v, buf_v, pbuf_v,
            tbuf_v, rows_v, *sems):
        gsems = sems[:NBUF]
        ssems = sems[NBUF:2 * NBUF]
        isem = sems[2 * NBUF]
        wid = lax.axis_index("s") * NC + lax.axis_index("c")

        for s2 in range(S):
            pltpu.make_async_copy(
                ids_hbm.at[s2, pl.ds(wid * W, W)], idx_v.at[s2], isem).start()
        pltpu.sync_copy(pe_hbm, pe_v)
        for bc in range(W // L):
            rows_v[bc, :] = lax.iota(jnp.int32, L) + bc * L
        for s2 in range(S):
            pltpu.make_async_copy(
                ids_hbm.at[s2, pl.ds(wid * W, W)], idx_v.at[s2], isem).wait()

        def gather(g, b):
            return pltpu.make_async_copy(
                table_hbm.at[idx_v.at[g]], buf_v.at[pl.ds(b * W, W)], gsems[b])

        def scatter(g, b):
            return pltpu.make_async_copy(
                tbuf_v.at[pl.ds(b * TE, TE)], out_hbm.at[g, :, wid], ssems[b])

        for g0 in range(LOOK):
            gather(g0, g0).start()

        def outer(i, carry):
            for b in range(NBUF):
                g = i * NBUF + b
                gather(g, b).wait()

                pec = [pe_v[g, pl.ds(c * L, L)] for c in range(E // L)]

                def add_row(r, c2):
                    for c in range(E // L):
                        sl = pl.ds(c * L, L)
                        pbuf_v[r, sl] = buf_v[b * W + r, sl] + pec[c]
                    return c2

                lax.fori_loop(0, W, add_row, 0)

                for bc in range(W // L):
                    rvec = rows_v[bc, :]

                    def xpose(te, c2, _bc=bc, _rvec=rvec):
                        for ep in range(8):
                            col = (_rvec * 0) + (te * 8 + ep)
                            v = plsc.load_gather(pbuf_v, [_rvec, col])
                            tbuf_v[b * TE + te, ep, pl.ds(_bc * L, L)] = v
                        return c2

                    lax.fori_loop(0, TE, xpose, 0)

                scatter(g, b).start()

                gn = g + LOOK
                nb = (b + LOOK) % NBUF

                @pl.when(gn < S)
                def _():
                    @pl.when(gn >= NBUF)
                    def _():
                        scatter(gn - NBUF, nb).wait()
                    gather(gn, nb).start()
            return carry

        lax.fori_loop(0, S // NBUF, outer, 0)

        for g0 in range(S - NBUF, S):
            scatter(g0, g0 % NBUF).wait()

    out5d = run(token_table, ids_t, pe)
    return out5d.transpose(2, 4, 0, 1, 3).reshape(B, S, E)


# speedup vs baseline: 2.1772x; 1.7425x over previous
import functools

import numpy as np
import jax
import jax.numpy as jnp
from jax import lax
from jax.experimental import pallas as pl
from jax.experimental.pallas import tpu as pltpu
from jax.experimental.pallas import tpu_sc as plsc


def _sinusoidal_pe(max_len, d):
    pos = np.arange(max_len, dtype=np.float32)[:, None]
    div = np.exp(np.arange(0, d, 2, dtype=np.float32) * (-np.log(10000.0) / d))
    pe = np.zeros((max_len, d), dtype=np.float32)
    pe[:, 0::2] = np.sin(pos * div)
    pe[:, 1::2] = np.cos(pos * div)
    return pe


def kernel(input, token_table):
    B, S = input.shape
    V, E = token_table.shape
    NC, NS = 2, 16
    NW = NC * NS
    L = 16
    W = B // NW
    TE = E // 8
    P = E + 1
    NBUF = 4
    LOOK = 2

    ids_t = input.astype(jnp.int32).T
    pe = jnp.asarray(_sinusoidal_pe(S, E))

    mesh = plsc.VectorSubcoreMesh(core_axis_name="c", subcore_axis_name="s")

    @functools.partial(
        pl.kernel,
        out_type=jax.ShapeDtypeStruct((S, TE, NW, 8, W), jnp.float32),
        mesh=mesh,
        compiler_params=pltpu.CompilerParams(
            use_tc_tiling_on_sc=False, needs_layout_passes=False),
        scratch_types=[
            pltpu.VMEM((S, W), jnp.int32),
            pltpu.VMEM((S, E), jnp.float32),
            pltpu.VMEM((NBUF * W, E), jnp.float32),
            pltpu.VMEM((W, P), jnp.float32),
            pltpu.VMEM((NBUF * TE, 8, W), jnp.float32),
            pltpu.VMEM((W // L, L), jnp.int32),
        ] + [pltpu.SemaphoreType.DMA] * (2 * NBUF + 1),
    )
    def run(table_hbm, ids_hbm, pe_hbm, out_hbm, idx_v, pe_v, buf_v, pbuf_v,
            tbuf_v, rows_v, *sems):
        gsems = sems[:NBUF]
        ssems = sems[NBUF:2 * NBUF]
        isem = sems[2 * NBUF]
        wid = lax.axis_index("s") * NC + lax.axis_index("c")

        for s2 in range(S):
            pltpu.make_async_copy(
                ids_hbm.at[s2, pl.ds(wid * W, W)], idx_v.at[s2], isem).start()
        pltpu.sync_copy(pe_hbm, pe_v)
        for bc in range(W // L):
            rows_v[bc, :] = lax.iota(jnp.int32, L) + bc * L
        for s2 in range(S):
            pltpu.make_async_copy(
                ids_hbm.at[s2, pl.ds(wid * W, W)], idx_v.at[s2], isem).wait()

        def gather(g, b):
            return pltpu.make_async_copy(
                table_hbm.at[idx_v.at[g]], buf_v.at[pl.ds(b * W, W)], gsems[b])

        def scatter(g, b):
            return pltpu.make_async_copy(
                tbuf_v.at[pl.ds(b * TE, TE)], out_hbm.at[g, :, wid], ssems[b])

        for g0 in range(LOOK):
            gather(g0, g0).start()

        def outer(i, carry):
            for b in range(NBUF):
                g = i * NBUF + b
                gather(g, b).wait()

                pec = [pe_v[g, pl.ds(c * L, L)] for c in range(E // L)]

                @plsc.parallel_loop(0, W, unroll=4)
                def add_row(r):
                    for c in range(E // L):
                        sl = pl.ds(c * L, L)
                        pbuf_v[r, sl] = buf_v[b * W + r, sl] + pec[c]

                for bc in range(W // L):
                    rvec = rows_v[bc, :]

                    @plsc.parallel_loop(0, TE, unroll=4)
                    def xpose(te, _bc=bc, _rvec=rvec):
                        for ep in range(8):
                            col = (_rvec * 0) + (te * 8 + ep)
                            v = plsc.load_gather(pbuf_v, [_rvec, col])
                            tbuf_v[b * TE + te, ep, pl.ds(_bc * L, L)] = v

                scatter(g, b).start()

                gn = g + LOOK
                nb = (b + LOOK) % NBUF

                @pl.when(gn < S)
                def _():
                    @pl.when(gn >= NBUF)
                    def _():
                        scatter(gn - NBUF, nb).wait()
                    gather(gn, nb).start()
            return carry

        lax.fori_loop(0, S // NBUF, outer, 0)

        for g0 in range(S - NBUF, S):
            scatter(g0, g0 % NBUF).wait()

    out5d = run(token_table, ids_t, pe)
    return out5d.transpose(2, 4, 0, 1, 3).reshape(B, S, E)


# speedup vs baseline: 2.1791x; 1.0009x over previous
import functools

import numpy as np
import jax
import jax.numpy as jnp
from jax import lax
from jax.experimental import pallas as pl
from jax.experimental.pallas import tpu as pltpu
from jax.experimental.pallas import tpu_sc as plsc


def _sinusoidal_pe(max_len, d):
    pos = np.arange(max_len, dtype=np.float32)[:, None]
    div = np.exp(np.arange(0, d, 2, dtype=np.float32) * (-np.log(10000.0) / d))
    pe = np.zeros((max_len, d), dtype=np.float32)
    pe[:, 0::2] = np.sin(pos * div)
    pe[:, 1::2] = np.cos(pos * div)
    return pe


def kernel(input, token_table):
    B, S = input.shape
    V, E = token_table.shape
    NC, NS = 2, 16
    NW = NC * NS
    L = 16
    W = B // NW
    TE = E // 8
    P = E + 1
    NBUF = 4
    LOOK = 2

    ids4 = (input.astype(jnp.int32)
            .reshape(NW, W, S // 8, 8).transpose(2, 0, 3, 1))
    pe = jnp.asarray(_sinusoidal_pe(S, E))

    mesh = plsc.VectorSubcoreMesh(core_axis_name="c", subcore_axis_name="s")

    @functools.partial(
        pl.kernel,
        out_type=jax.ShapeDtypeStruct((S, TE, NW, 8, W), jnp.float32),
        mesh=mesh,
        compiler_params=pltpu.CompilerParams(
            use_tc_tiling_on_sc=False, needs_layout_passes=False),
        scratch_types=[
            pltpu.VMEM((S // 8, 8, W), jnp.int32),
            pltpu.VMEM((S, E), jnp.float32),
            pltpu.VMEM((NBUF * W, E), jnp.float32),
            pltpu.VMEM((W, P), jnp.float32),
            pltpu.VMEM((NBUF * TE, 8, W), jnp.float32),
            pltpu.VMEM((W // L, L), jnp.int32),
        ] + [pltpu.SemaphoreType.DMA] * (2 * NBUF),
    )
    def run(table_hbm, ids_hbm, pe_hbm, out_hbm, idx_v, pe_v, buf_v, pbuf_v,
            tbuf_v, rows_v, *sems):
        gsems = sems[:NBUF]
        ssems = sems[NBUF:2 * NBUF]
        wid = lax.axis_index("s") * NC + lax.axis_index("c")

        pltpu.sync_copy(ids_hbm.at[:, wid], idx_v)
        pltpu.sync_copy(pe_hbm, pe_v)
        for bc in range(W // L):
            rows_v[bc, :] = lax.iota(jnp.int32, L) + bc * L

        def gather(g, b):
            return pltpu.make_async_copy(
                table_hbm.at[idx_v.at[g // 8, g % 8]],
                buf_v.at[pl.ds(b * W, W)], gsems[b])

        def scatter(g, b):
            return pltpu.make_async_copy(
                tbuf_v.at[pl.ds(b * TE, TE)], out_hbm.at[g, :, wid], ssems[b])

        for g0 in range(LOOK):
            gather(g0, g0).start()

        def outer(i, carry):
            for b in range(NBUF):
                g = i * NBUF + b
                gather(g, b).wait()

                pec = [pe_v[g, pl.ds(c * L, L)] for c in range(E // L)]

                @plsc.parallel_loop(0, W, unroll=8)
                def add_row(r):
                    for c in range(E // L):
                        sl = pl.ds(c * L, L)
                        pbuf_v[r, sl] = buf_v[b * W + r, sl] + pec[c]

                for bc in range(W // L):
                    rvec = rows_v[bc, :]

                    @plsc.parallel_loop(0, TE, unroll=4)
                    def xpose(te, _bc=bc, _rvec=rvec):
                        for ep in range(8):
                            col = (_rvec * 0) + (te * 8 + ep)
                            v = plsc.load_gather(pbuf_v, [_rvec, col])
                            tbuf_v[b * TE + te, ep, pl.ds(_bc * L, L)] = v

                scatter(g, b).start()

                gn = g + LOOK
                nb = (b + LOOK) % NBUF

                @pl.when(gn < S)
                def _():
                    @pl.when(gn >= NBUF)
                    def _():
                        scatter(gn - NBUF, nb).wait()
                    gather(gn, nb).start()
            return carry

        lax.fori_loop(0, S // NBUF, outer, 0)

        for g0 in range(S - NBUF, S):
            scatter(g0, g0 % NBUF).wait()

    out5d = run(token_table, ids4, pe)
    return out5d.transpose(2, 4, 0, 1, 3).reshape(B, S, E)
